# baseline (speedup 1.0000x reference)
import functools

import jax
import jax.numpy as jnp
from jax.experimental import pallas as pl
from jax.experimental.pallas import tpu as pltpu


def _roll(z, s):
    return pltpu.roll(z, s % z.shape[-1], axis=1)


def _reflect_taps(z, *, HO, WO, ho, wo):
    left = _roll(z, 1)
    right = _roll(z, -1)
    cols = [jnp.where(wo == 0, right, left),
            z,
            jnp.where(wo == WO - 1, left, right)]
    up, mid, dn = [], [], []
    for w in cols:
        above = _roll(w, WO)
        below = _roll(w, -WO)
        up.append(jnp.where(ho == 0, below, above))
        mid.append(w)
        dn.append(jnp.where(ho == HO - 1, above, below))
    return up + mid + dn


def _step_kernel(xp_ref, wd_ref, bd_ref, wg_ref,
                 bl_ref, out_ref, h_ref, c_ref, *, CIN, CO, HO, WO):
    t = pl.program_id(0)

    @pl.when(t == 0)
    def _():
        h_ref[...] = jnp.zeros_like(h_ref)
        c_ref[...] = jnp.zeros_like(c_ref)

    Mb = xp_ref.shape[-1]
    m = jax.lax.broadcasted_iota(jnp.int32, (1, Mb), 1)
    wo = m % WO
    ho = (m // WO) % HO
    mh = ho > 0
    mw = wo > 0

    def unpack(u):
        lo = jax.lax.bitcast_convert_type(u << 16, jnp.float32)
        hi = jax.lax.bitcast_convert_type(u & jnp.uint32(0xFFFF0000),
                                          jnp.float32)
        return lo.astype(jnp.bfloat16), hi.astype(jnp.bfloat16)

    p00, p01 = unpack(xp_ref[0 * CIN:1 * CIN])
    p10, p11 = unpack(xp_ref[1 * CIN:2 * CIN])
    xcols = jnp.concatenate([
        jnp.where(mh & mw, _roll(p11, WO + 1), 0),
        jnp.where(mh, _roll(p10, WO), 0),
        jnp.where(mh, _roll(p11, WO), 0),
        jnp.where(mw, _roll(p01, 1), 0),
        p00,
        p01,
        jnp.where(mw, _roll(p11, 1), 0),
        p10,
        p11,
    ], axis=0)

    y = (jnp.dot(wd_ref[...], xcols, preferred_element_type=jnp.float32)
         + bd_ref[...])

    y_taps = _reflect_taps(y.astype(jnp.bfloat16), HO=HO, WO=WO, ho=ho, wo=wo)
    h_taps = _reflect_taps(h_ref[...].astype(jnp.bfloat16),
                           HO=HO, WO=WO, ho=ho, wo=wo)
    cols = jnp.concatenate(y_taps + h_taps, axis=0)
    gates = (jnp.dot(wg_ref[...], cols, preferred_element_type=jnp.float32)
             + bl_ref[...])

    i_g = jax.nn.sigmoid(gates[0:CO])
    f_g = jax.nn.sigmoid(gates[CO:2 * CO])
    o_g = jax.nn.sigmoid(gates[2 * CO:3 * CO])
    g_g = jnp.tanh(gates[3 * CO:4 * CO])

    c_new = f_g * c_ref[...] + i_g * g_g
    h_new = o_g * jnp.tanh(c_new)
    c_ref[...] = c_new
    h_ref[...] = h_new
    B = out_ref.shape[0]
    MB = Mb // B
    for b in range(B):
        out_ref[b] = h_new[:, b * MB:(b + 1) * MB]


def kernel(x_btchw, w_down, b_down, w_lstm, b_lstm):
    B, T, CIN, H, W = x_btchw.shape
    CH, _, K, _ = w_down.shape
    CO = w_lstm.shape[0] // 4
    S, padding = 2, 1
    HO = (H + 2 * padding - K) // S + 1
    WO = (W + 2 * padding - K) // S + 1
    assert K == 3 and H == S * HO and W == S * WO
    MB = HO * WO
    M = B * MB
    f32, bf16 = jnp.float32, jnp.bfloat16

    xb = x_btchw.astype(bf16).reshape(B, T, CIN, H, WO, 2)
    xu = jax.lax.bitcast_convert_type(xb, jnp.uint32)
    xq = xu.reshape(B, T, CIN, HO, 2, WO)
    xq = jnp.transpose(xq, (1, 4, 2, 0, 3, 5)).reshape(T, 2 * CIN, M)

    wd = jnp.transpose(w_down, (0, 2, 3, 1)).reshape(CH, K * K * CIN)
    wl = jnp.transpose(w_lstm, (0, 2, 3, 1)).reshape(4 * CO, K * K, CH + CO)
    wg = jnp.concatenate([wl[:, :, :CH].reshape(4 * CO, K * K * CH),
                          wl[:, :, CH:].reshape(4 * CO, K * K * CO)], axis=1)
    wd, wg = wd.astype(bf16), wg.astype(bf16)
    bd = b_down.reshape(CH, 1).astype(f32)
    bl = b_lstm.reshape(4 * CO, 1).astype(f32)

    out = pl.pallas_call(
        functools.partial(_step_kernel, CIN=CIN, CO=CO, HO=HO, WO=WO),
        out_shape=jax.ShapeDtypeStruct((B, T, CO, MB), f32),
        grid=(T,),
        in_specs=[
            pl.BlockSpec((None, 2 * CIN, M), lambda t: (t, 0, 0)),
            pl.BlockSpec((CH, K * K * CIN), lambda t: (0, 0)),
            pl.BlockSpec((CH, 1), lambda t: (0, 0)),
            pl.BlockSpec((4 * CO, K * K * (CH + CO)), lambda t: (0, 0)),
            pl.BlockSpec((4 * CO, 1), lambda t: (0, 0)),
        ],
        out_specs=pl.BlockSpec((B, None, CO, MB), lambda t: (0, t, 0, 0)),
        scratch_shapes=[
            pltpu.VMEM((CO, M), f32),
            pltpu.VMEM((CO, M), f32),
        ],
        compiler_params=pltpu.CompilerParams(
            dimension_semantics=("arbitrary",)),
    )(xq, wd, bd, wg, bl)

    return out.reshape(B, T, CO, HO, WO)


# speedup vs baseline: 1.0756x; 1.0756x over previous
import functools

import jax
import jax.numpy as jnp
from jax.experimental import pallas as pl
from jax.experimental.pallas import tpu as pltpu


def _roll(z, s):
    return pltpu.roll(z, s % z.shape[-1], axis=1)


def _reflect_taps(z, *, HO, WO, ho, wo):
    left = _roll(z, 1)
    right = _roll(z, -1)
    cols = [jnp.where(wo == 0, right, left),
            z,
            jnp.where(wo == WO - 1, left, right)]
    up, mid, dn = [], [], []
    for w in cols:
        above = _roll(w, WO)
        below = _roll(w, -WO)
        up.append(jnp.where(ho == 0, below, above))
        mid.append(w)
        dn.append(jnp.where(ho == HO - 1, above, below))
    return up + mid + dn


def _step_kernel(xp_ref, wd_ref, bd_ref, wg_ref,
                 bl_ref, out_ref, h_ref, c_ref, *, CIN, CO, HO, WO):
    t = pl.program_id(0)

    @pl.when(t == 0)
    def _():
        h_ref[...] = jnp.zeros_like(h_ref)
        c_ref[...] = jnp.zeros_like(c_ref)

    Mb = xp_ref.shape[-1]
    m = jax.lax.broadcasted_iota(jnp.int32, (1, Mb), 1)
    wo = m % WO
    ho = (m // WO) % HO
    mh = ho > 0
    mw = wo > 0

    p00 = xp_ref[0 * CIN:1 * CIN]
    p01 = xp_ref[1 * CIN:2 * CIN]
    p10 = xp_ref[2 * CIN:3 * CIN]
    p11 = xp_ref[3 * CIN:4 * CIN]
    xcols = jnp.concatenate([
        jnp.where(mh & mw, _roll(p11, WO + 1), 0),
        jnp.where(mh, _roll(p10, WO), 0),
        jnp.where(mh, _roll(p11, WO), 0),
        jnp.where(mw, _roll(p01, 1), 0),
        p00,
        p01,
        jnp.where(mw, _roll(p11, 1), 0),
        p10,
        p11,
    ], axis=0)

    y = (jnp.dot(wd_ref[...], xcols, preferred_element_type=jnp.float32)
         + bd_ref[...])

    y_taps = _reflect_taps(y.astype(jnp.bfloat16), HO=HO, WO=WO, ho=ho, wo=wo)
    h_taps = _reflect_taps(h_ref[...].astype(jnp.bfloat16),
                           HO=HO, WO=WO, ho=ho, wo=wo)
    cols = jnp.concatenate(y_taps + h_taps, axis=0)
    gates = (jnp.dot(wg_ref[...], cols, preferred_element_type=jnp.float32)
             + bl_ref[...])

    i_g = jax.nn.sigmoid(gates[0:CO])
    f_g = jax.nn.sigmoid(gates[CO:2 * CO])
    o_g = jax.nn.sigmoid(gates[2 * CO:3 * CO])
    g_g = jnp.tanh(gates[3 * CO:4 * CO])

    c_new = f_g * c_ref[...] + i_g * g_g
    h_new = o_g * jnp.tanh(c_new)
    c_ref[...] = c_new
    h_ref[...] = h_new
    B = out_ref.shape[0]
    MB = Mb // B
    for b in range(B):
        out_ref[b] = h_new[:, b * MB:(b + 1) * MB]


def kernel(x_btchw, w_down, b_down, w_lstm, b_lstm):
    B, T, CIN, H, W = x_btchw.shape
    CH, _, K, _ = w_down.shape
    CO = w_lstm.shape[0] // 4
    S, padding = 2, 1
    HO = (H + 2 * padding - K) // S + 1
    WO = (W + 2 * padding - K) // S + 1
    assert K == 3 and H == S * HO and W == S * WO
    MB = HO * WO
    M = B * MB
    f32, bf16 = jnp.float32, jnp.bfloat16

    xq = x_btchw.reshape(B, T, CIN, HO, 2, WO, 2)
    xq = jnp.transpose(xq, (1, 4, 6, 2, 0, 3, 5)).reshape(T, 4 * CIN, M)
    xq = xq.astype(bf16)

    wd = jnp.transpose(w_down, (0, 2, 3, 1)).reshape(CH, K * K * CIN)
    wl = jnp.transpose(w_lstm, (0, 2, 3, 1)).reshape(4 * CO, K * K, CH + CO)
    wg = jnp.concatenate([wl[:, :, :CH].reshape(4 * CO, K * K * CH),
                          wl[:, :, CH:].reshape(4 * CO, K * K * CO)], axis=1)
    wd, wg = wd.astype(bf16), wg.astype(bf16)
    bd = b_down.reshape(CH, 1).astype(f32)
    bl = b_lstm.reshape(4 * CO, 1).astype(f32)

    out = pl.pallas_call(
        functools.partial(_step_kernel, CIN=CIN, CO=CO, HO=HO, WO=WO),
        out_shape=jax.ShapeDtypeStruct((B, T, CO, MB), f32),
        grid=(T,),
        in_specs=[
            pl.BlockSpec((None, 4 * CIN, M), lambda t: (t, 0, 0)),
            pl.BlockSpec((CH, K * K * CIN), lambda t: (0, 0)),
            pl.BlockSpec((CH, 1), lambda t: (0, 0)),
            pl.BlockSpec((4 * CO, K * K * (CH + CO)), lambda t: (0, 0)),
            pl.BlockSpec((4 * CO, 1), lambda t: (0, 0)),
        ],
        out_specs=pl.BlockSpec((B, None, CO, MB), lambda t: (0, t, 0, 0)),
        scratch_shapes=[
            pltpu.VMEM((CO, M), f32),
            pltpu.VMEM((CO, M), f32),
        ],
        compiler_params=pltpu.CompilerParams(
            dimension_semantics=("arbitrary",)),
    )(xq, wd, bd, wg, bl)

    return out.reshape(B, T, CO, HO, WO)


# speedup vs baseline: 1.0996x; 1.0222x over previous
import functools

import jax
import jax.numpy as jnp
from jax.experimental import pallas as pl
from jax.experimental.pallas import tpu as pltpu


def _roll(z, s):
    return pltpu.roll(z, s % z.shape[-1], axis=1)


def _reflect_taps(z, *, HO, WO, ho, wo):
    left = _roll(z, 1)
    right = _roll(z, -1)
    cols = [jnp.where(wo == 0, right, left),
            z,
            jnp.where(wo == WO - 1, left, right)]
    up, mid, dn = [], [], []
    for w in cols:
        above = _roll(w, WO)
        below = _roll(w, -WO)
        up.append(jnp.where(ho == 0, below, above))
        mid.append(w)
        dn.append(jnp.where(ho == HO - 1, above, below))
    return up + mid + dn


def _step_kernel(xp_ref, wd_ref, wg_ref,
                 bl_ref, out_ref, h_ref, c_ref, *, CIN, CO, HO, WO):
    t = pl.program_id(0)

    @pl.when(t == 0)
    def _():
        h_ref[...] = jnp.zeros_like(h_ref)
        c_ref[...] = jnp.zeros_like(c_ref)

    Mb = xp_ref.shape[-1]
    m = jax.lax.broadcasted_iota(jnp.int32, (1, Mb), 1)
    wo = m % WO
    ho = (m // WO) % HO
    mh = ho > 0
    mw = wo > 0

    p00 = xp_ref[0 * CIN:1 * CIN]
    p01 = xp_ref[1 * CIN:2 * CIN]
    p10 = xp_ref[2 * CIN:3 * CIN]
    p11 = xp_ref[3 * CIN:4 * CIN]
    xcols = jnp.concatenate([
        jnp.where(mh & mw, _roll(p11, WO + 1), 0),
        jnp.where(mh, _roll(p10, WO), 0),
        jnp.where(mh, _roll(p11, WO), 0),
        jnp.where(mw, _roll(p01, 1), 0),
        p00,
        p01,
        jnp.where(mw, _roll(p11, 1), 0),
        p10,
        p11,
    ], axis=0)

    y = jnp.dot(wd_ref[...], xcols,
                preferred_element_type=jnp.float32)

    yh = jnp.concatenate([y.astype(jnp.bfloat16), h_ref[...]], axis=0)
    yh_taps = _reflect_taps(yh, HO=HO, WO=WO, ho=ho, wo=wo)
    cols = jnp.concatenate(yh_taps, axis=0)
    gates = (jnp.dot(wg_ref[...], cols, preferred_element_type=jnp.float32)
             + bl_ref[...])

    i_g = jax.nn.sigmoid(gates[0:CO])
    f_g = jax.nn.sigmoid(gates[CO:2 * CO])
    o_g = jax.nn.sigmoid(gates[2 * CO:3 * CO])
    g_g = jnp.tanh(gates[3 * CO:4 * CO])

    c_new = f_g * c_ref[...] + i_g * g_g
    h_new = o_g * jnp.tanh(c_new)
    c_ref[...] = c_new
    h_ref[...] = h_new.astype(jnp.bfloat16)
    B = out_ref.shape[0]
    MB = Mb // B
    for b in range(B):
        out_ref[b] = h_new[:, b * MB:(b + 1) * MB]


def kernel(x_btchw, w_down, b_down, w_lstm, b_lstm):
    B, T, CIN, H, W = x_btchw.shape
    CH, _, K, _ = w_down.shape
    CO = w_lstm.shape[0] // 4
    S, padding = 2, 1
    HO = (H + 2 * padding - K) // S + 1
    WO = (W + 2 * padding - K) // S + 1
    assert K == 3 and H == S * HO and W == S * WO
    MB = HO * WO
    M = B * MB
    f32, bf16 = jnp.float32, jnp.bfloat16

    xq = x_btchw.reshape(B, T, CIN, HO, 2, WO, 2)
    xq = jnp.transpose(xq, (1, 4, 6, 2, 0, 3, 5)).reshape(T, 4 * CIN, M)
    xq = xq.astype(bf16)

    wd = jnp.transpose(w_down, (0, 2, 3, 1)).reshape(CH, K * K * CIN)
    wg = jnp.transpose(w_lstm, (0, 2, 3, 1)).reshape(4 * CO,
                                                     K * K * (CH + CO))
    wd, wg = wd.astype(bf16), wg.astype(bf16)
    wl3 = jnp.transpose(w_lstm, (0, 2, 3, 1)).reshape(4 * CO, K * K, CH + CO)
    bl = (b_lstm.astype(f32)
          + wl3[:, :, :CH].sum(1).astype(f32) @ b_down.astype(f32))
    bl = bl.reshape(4 * CO, 1)

    out = pl.pallas_call(
        functools.partial(_step_kernel, CIN=CIN, CO=CO, HO=HO, WO=WO),
        out_shape=jax.ShapeDtypeStruct((B, T, CO, MB), f32),
        grid=(T,),
        in_specs=[
            pl.BlockSpec((None, 4 * CIN, M), lambda t: (t, 0, 0)),
            pl.BlockSpec((CH, K * K * CIN), lambda t: (0, 0)),
            pl.BlockSpec((4 * CO, K * K * (CH + CO)), lambda t: (0, 0)),
            pl.BlockSpec((4 * CO, 1), lambda t: (0, 0)),
        ],
        out_specs=pl.BlockSpec((B, None, CO, MB), lambda t: (0, t, 0, 0)),
        scratch_shapes=[
            pltpu.VMEM((CO, M), bf16),
            pltpu.VMEM((CO, M), f32),
        ],
        compiler_params=pltpu.CompilerParams(
            dimension_semantics=("arbitrary",)),
    )(xq, wd, wg, bl)

    return out.reshape(B, T, CO, HO, WO)


# speedup vs baseline: 1.3905x; 1.2645x over previous
import functools

import jax
import jax.numpy as jnp
from jax.experimental import pallas as pl
from jax.experimental.pallas import tpu as pltpu


def _roll(z, s):
    return pltpu.roll(z, s % z.shape[-1], axis=1)


def _reflect_taps(z, *, HO, WO, ho, wo):
    left = _roll(z, 1)
    right = _roll(z, -1)
    cols = [jnp.where(wo == 0, right, left),
            z,
            jnp.where(wo == WO - 1, left, right)]
    up, mid, dn = [], [], []
    for w in cols:
        above = _roll(w, WO)
        below = _roll(w, -WO)
        up.append(jnp.where(ho == 0, below, above))
        mid.append(w)
        dn.append(jnp.where(ho == HO - 1, above, below))
    return up + mid + dn


def _step_kernel(xp_ref, wd_ref, wg_ref,
                 bl_ref, out_ref, h_ref, c_ref, *, CIN, CO, HO, WO):
    t = pl.program_id(0)

    @pl.when(t == 0)
    def _():
        h_ref[...] = jnp.zeros_like(h_ref)
        c_ref[...] = jnp.zeros_like(c_ref)

    Mb = xp_ref.shape[-1]
    m = jax.lax.broadcasted_iota(jnp.int32, (1, Mb), 1)
    wo = m % WO
    ho = (m // WO) % HO
    mh = ho > 0
    mw = wo > 0

    p00 = xp_ref[0 * CIN:1 * CIN]
    p01 = xp_ref[1 * CIN:2 * CIN]
    p10 = xp_ref[2 * CIN:3 * CIN]
    p11 = xp_ref[3 * CIN:4 * CIN]
    xcols = jnp.concatenate([
        jnp.where(mh & mw, _roll(p11, WO + 1), 0),
        jnp.where(mh, _roll(p10, WO), 0),
        jnp.where(mh, _roll(p11, WO), 0),
        jnp.where(mw, _roll(p01, 1), 0),
        p00,
        p01,
        jnp.where(mw, _roll(p11, 1), 0),
        p10,
        p11,
    ], axis=0)

    y = jnp.dot(wd_ref[...], xcols,
                preferred_element_type=jnp.float32)

    yh = jnp.concatenate([y.astype(jnp.bfloat16), h_ref[...]], axis=0)
    yh_taps = _reflect_taps(yh, HO=HO, WO=WO, ho=ho, wo=wo)
    cols = jnp.concatenate(yh_taps, axis=0)
    gates = (jnp.dot(wg_ref[...], cols, preferred_element_type=jnp.float32)
             + bl_ref[...])

    i_g = jax.nn.sigmoid(gates[0:CO])
    f_g = jax.nn.sigmoid(gates[CO:2 * CO])
    o_g = jax.nn.sigmoid(gates[2 * CO:3 * CO])
    g_g = jnp.tanh(gates[3 * CO:4 * CO])

    c_new = f_g * c_ref[...] + i_g * g_g
    h_new = o_g * jnp.tanh(c_new)
    c_ref[...] = c_new
    h_ref[...] = h_new.astype(jnp.bfloat16)
    B = out_ref.shape[0]
    MB = Mb // B
    for b in range(B):
        out_ref[b] = h_new[:, b * MB:(b + 1) * MB]


def kernel(x_btchw, w_down, b_down, w_lstm, b_lstm):
    B, T, CIN, H, W = x_btchw.shape
    CH, _, K, _ = w_down.shape
    CO = w_lstm.shape[0] // 4
    S, padding = 2, 1
    HO = (H + 2 * padding - K) // S + 1
    WO = (W + 2 * padding - K) // S + 1
    assert K == 3 and H == S * HO and W == S * WO
    MB = HO * WO
    M = B * MB
    f32, bf16 = jnp.float32, jnp.bfloat16

    eye = jnp.eye(CIN, dtype=bf16)
    filt = jnp.zeros((4 * CIN, CIN, 2, 2), bf16)
    for py in (0, 1):
        for px in (0, 1):
            filt = filt.at[(py * 2 + px) * CIN:(py * 2 + px + 1) * CIN,
                           :, py, px].set(eye)
    xq = jax.lax.conv_general_dilated(
        x_btchw.reshape(B * T, CIN, H, W).astype(bf16), filt,
        window_strides=(2, 2), padding="VALID",
        dimension_numbers=("NCHW", "OIHW", "NCHW"),
        preferred_element_type=bf16)
    xq = xq.reshape(B, T, 4 * CIN, HO, WO)
    xq = jnp.transpose(xq, (1, 2, 0, 3, 4)).reshape(T, 4 * CIN, M)

    wd = jnp.transpose(w_down, (0, 2, 3, 1)).reshape(CH, K * K * CIN)
    wg = jnp.transpose(w_lstm, (0, 2, 3, 1)).reshape(4 * CO,
                                                     K * K * (CH + CO))
    wd, wg = wd.astype(bf16), wg.astype(bf16)
    wl3 = jnp.transpose(w_lstm, (0, 2, 3, 1)).reshape(4 * CO, K * K, CH + CO)
    bl = (b_lstm.astype(f32)
          + wl3[:, :, :CH].sum(1).astype(f32) @ b_down.astype(f32))
    bl = bl.reshape(4 * CO, 1)

    out = pl.pallas_call(
        functools.partial(_step_kernel, CIN=CIN, CO=CO, HO=HO, WO=WO),
        out_shape=jax.ShapeDtypeStruct((B, T, CO, MB), f32),
        grid=(T,),
        in_specs=[
            pl.BlockSpec((None, 4 * CIN, M), lambda t: (t, 0, 0)),
            pl.BlockSpec((CH, K * K * CIN), lambda t: (0, 0)),
            pl.BlockSpec((4 * CO, K * K * (CH + CO)), lambda t: (0, 0)),
            pl.BlockSpec((4 * CO, 1), lambda t: (0, 0)),
        ],
        out_specs=pl.BlockSpec((B, None, CO, MB), lambda t: (0, t, 0, 0)),
        scratch_shapes=[
            pltpu.VMEM((CO, M), bf16),
            pltpu.VMEM((CO, M), f32),
        ],
        compiler_params=pltpu.CompilerParams(
            dimension_semantics=("arbitrary",)),
    )(xq, wd, wg, bl)

    return out.reshape(B, T, CO, HO, WO)
